# Initial kernel scaffold; baseline (speedup 1.0000x reference)
import jax
import jax.numpy as jnp
from jax.experimental import pallas as pl


def kernel(x, edge_index, W1_rel, b1_rel, W1_root, W2_rel, b2_rel, W2_root, Wl, bl, Wd1, bd1, Wo1, bo1, Wd2, bd2, Wo2, bo2, Wd3, bd3, Wo3, bo3):
    raise NotImplementedError("write your pallas kernel here")


# speedup vs baseline: 3.2540x; 3.2540x over previous
import functools

import jax
import jax.numpy as jnp
from jax import lax
from jax.experimental import pallas as pl
from jax.experimental.pallas import tpu as pltpu
from jax.experimental.pallas import tpu_sc as plsc

NC = 2
NS = 16
CH = 128
ACC_ROWS = 10240


def _sc_scatter(table, src_idx, dst_idx, n_out):
    width = table.shape[1]
    nch = src_idx.shape[2]
    rows_per = ACC_ROWS // NS
    zeros = jnp.zeros((ACC_ROWS, width), jnp.float32)
    mesh = plsc.VectorSubcoreMesh(core_axis_name="c", subcore_axis_name="s")

    @functools.partial(
        pl.kernel,
        mesh=mesh,
        out_type=jax.ShapeDtypeStruct((NC, ACC_ROWS, width), jnp.float32),
        scratch_types=[
            pltpu.VMEM((nch, CH), jnp.int32),
            pltpu.VMEM((nch, CH), jnp.int32),
            pltpu.VMEM((CH, width), jnp.float32),
            pltpu.VMEM_SHARED((ACC_ROWS, width), jnp.float32),
            pltpu.SemaphoreType.DMA,
        ],
    )
    def k(table_hbm, src_hbm, dst_hbm, z_hbm, out_hbm, src_v, dst_v, rows_v,
          acc, sem):
        c = lax.axis_index("c")
        s = lax.axis_index("s")
        pltpu.sync_copy(z_hbm.at[pl.ds(s * rows_per, rows_per)],
                        acc.at[pl.ds(s * rows_per, rows_per)])
        pltpu.sync_copy(src_hbm.at[c, s], src_v)
        pltpu.sync_copy(dst_hbm.at[c, s], dst_v)
        plsc.subcore_barrier()

        def body(j, carry):
            pltpu.async_copy(table_hbm.at[src_v.at[j]], rows_v, sem).wait()
            pltpu.sync_copy(rows_v, acc.at[dst_v.at[j]], add=True)
            return carry

        lax.fori_loop(0, nch, body, 0)
        plsc.subcore_barrier()
        pltpu.sync_copy(acc.at[pl.ds(s * rows_per, rows_per)],
                        out_hbm.at[c, pl.ds(s * rows_per, rows_per)])

    return k(table, src_idx, dst_idx, zeros)


def _tc_encoder1(aggp, x16, wr, wx, b1):
    n = x16.shape[0]
    bn = 1000

    def body(agg_ref, x_ref, wr_ref, wx_ref, b_ref, out_ref):
        a = agg_ref[0] + agg_ref[1]
        h = jnp.dot(a, wr_ref[...], preferred_element_type=jnp.float32)
        h = h + jnp.dot(x_ref[...], wx_ref[...],
                        preferred_element_type=jnp.float32)
        h = jnp.maximum(h + b_ref[...], 0.0)
        out_ref[0] = h[:, :128]
        out_ref[1] = h[:, 128:]

    return pl.pallas_call(
        body,
        grid=(n // bn,),
        in_specs=[
            pl.BlockSpec((2, bn, 128), lambda i: (0, i, 0)),
            pl.BlockSpec((bn, 16), lambda i: (i, 0)),
            pl.BlockSpec((128, 256), lambda i: (0, 0)),
            pl.BlockSpec((16, 256), lambda i: (0, 0)),
            pl.BlockSpec((1, 256), lambda i: (0, 0)),
        ],
        out_specs=pl.BlockSpec((2, bn, 128), lambda i: (0, i, 0)),
        out_shape=jax.ShapeDtypeStruct((2, n, 128), jnp.float32),
    )(aggp, x16, wr, wx, b1)


def _tc_decoder(agg2, h12, w2, b2, wlt, bl, wd, bd, wo, bo):
    n = h12.shape[1]
    bn = 1000

    def body(agg_ref, h1_ref, w2_ref, b2_ref, wl_ref, bl_ref, wd_ref, bd_ref,
             wo_ref, bo_ref, out_ref):
        h2 = jnp.dot(agg_ref[0], w2_ref[0], preferred_element_type=jnp.float32)
        h2 = h2 + jnp.dot(agg_ref[1], w2_ref[1],
                          preferred_element_type=jnp.float32)
        h2 = h2 + jnp.dot(h1_ref[0], w2_ref[2],
                          preferred_element_type=jnp.float32)
        h2 = h2 + jnp.dot(h1_ref[1], w2_ref[3],
                          preferred_element_type=jnp.float32)
        h2 = jnp.maximum(h2 + b2_ref[...], 0.0)
        h3 = jnp.dot(h2, wl_ref[...],
                     preferred_element_type=jnp.float32) + bl_ref[...]
        g = jnp.dot(h3, wd_ref[...],
                    preferred_element_type=jnp.float32) + bd_ref[...]
        g = jnp.where(g > 0.0, g, jnp.exp(jnp.minimum(g, 0.0)) - 1.0)
        out_ref[...] = jnp.dot(
            g, wo_ref[...], preferred_element_type=jnp.float32) + bo_ref[...]

    return pl.pallas_call(
        body,
        grid=(n // bn,),
        in_specs=[
            pl.BlockSpec((2, bn, 128), lambda i: (0, i, 0)),
            pl.BlockSpec((2, bn, 128), lambda i: (0, i, 0)),
            pl.BlockSpec((4, 128, 256), lambda i: (0, 0, 0)),
            pl.BlockSpec((1, 256), lambda i: (0, 0)),
            pl.BlockSpec((256, 384), lambda i: (0, 0)),
            pl.BlockSpec((1, 384), lambda i: (0, 0)),
            pl.BlockSpec((384, 192), lambda i: (0, 0)),
            pl.BlockSpec((1, 192), lambda i: (0, 0)),
            pl.BlockSpec((192, 8), lambda i: (0, 0)),
            pl.BlockSpec((1, 8), lambda i: (0, 0)),
        ],
        out_specs=pl.BlockSpec((bn, 8), lambda i: (i, 0)),
        out_shape=jax.ShapeDtypeStruct((n, 8), jnp.float32),
    )(agg2, h12, w2, b2, wlt, bl, wd, bd, wo, bo)


def kernel(x, edge_index, W1_rel, b1_rel, W1_root, W2_rel, b2_rel, W2_root,
           Wl, bl, Wd1, bd1, Wo1, bo1, Wd2, bd2, Wo2, bo2, Wd3, bd3, Wo3, bo3):
    n = x.shape[0]
    e = edge_index.shape[1]
    src = edge_index[0]
    dst = edge_index[1]

    nch1 = -(-e // (NC * NS * CH))
    e1 = NC * NS * nch1 * CH
    src1 = jnp.concatenate(
        [src, jnp.zeros((e1 - e,), jnp.int32)]).reshape(NC, NS, nch1, CH)
    dst1 = jnp.concatenate(
        [dst, jnp.full((e1 - e,), n, jnp.int32)]).reshape(NC, NS, nch1, CH)

    nch2 = -(-e // (NS * CH))
    e2 = NS * nch2 * CH
    srcp = jnp.concatenate(
        [src, jnp.zeros((e2 - e,), jnp.int32)]).reshape(NS, nch2, CH)
    src2 = jnp.stack([srcp, srcp + n])
    dstp = jnp.concatenate(
        [dst, jnp.full((e2 - e,), n, jnp.int32)]).reshape(NS, nch2, CH)
    dst2 = jnp.stack([dstp, dstp])

    x16 = jnp.pad(x, ((0, 0), (0, 16 - x.shape[1])))
    x128 = jnp.pad(x, ((0, 0), (0, 128 - x.shape[1])))

    aggp1 = _sc_scatter(x128, src1, dst1, n)

    wr1 = jnp.pad(W1_rel.T, ((0, 128 - W1_rel.shape[1]), (0, 0)))
    wx1 = jnp.pad(W1_root.T, ((0, 16 - W1_root.shape[1]), (0, 0)))
    h12 = _tc_encoder1(aggp1, x16, wr1, wx1, b1_rel[None])

    agg2 = _sc_scatter(h12.reshape(2 * n, 128), src2, dst2, n)

    w2 = jnp.stack([W2_rel.T[:128], W2_rel.T[128:],
                    W2_root.T[:128], W2_root.T[128:]])
    wd = (jnp.zeros((384, 192), jnp.float32)
          .at[0:128, 0:64].set(Wd1.T)
          .at[128:256, 64:128].set(Wd2.T)
          .at[256:384, 128:192].set(Wd3.T))
    bd = jnp.concatenate([bd1, bd2, bd3])
    wo = (jnp.zeros((192, 8), jnp.float32)
          .at[0:64, 0].set(Wo1[0])
          .at[64:128, 1].set(Wo2[0])
          .at[128:192, 2].set(Wo3[0]))
    bo = jnp.concatenate([bo1, bo2, bo3, jnp.zeros((5,), jnp.float32)])

    out8 = _tc_decoder(agg2, h12, w2, b2_rel[None], Wl.T, bl[None],
                       wd, bd[None], wo, bo[None])
    return out8[:, :3].reshape(n, 3, 1)
